# speedup vs baseline: 6.7406x; 6.7406x over previous
import functools

import jax
import jax.numpy as jnp
from jax import lax
from jax.experimental import pallas as pl
from jax.experimental.pallas import tpu as pltpu
from jax.experimental.pallas import tpu_sc as plsc

NC = 2
NS = 16
NW = NC * NS
CH = 128


def _sc_aggregate(n_pad, d, epw):
    nchunk = epw // CH
    rows_per_tile = n_pad // NS
    mesh = plsc.VectorSubcoreMesh(core_axis_name="c", subcore_axis_name="s")

    @functools.partial(
        pl.kernel,
        mesh=mesh,
        out_type=jax.ShapeDtypeStruct((NC, n_pad, d), jnp.float32),
        scratch_types=[
            pltpu.VMEM((CH,), jnp.int32),
            pltpu.VMEM((CH,), jnp.int32),
            pltpu.VMEM((CH, d), jnp.float32),
            pltpu.VMEM_SHARED((n_pad, d), jnp.float32),
            pltpu.SemaphoreType.DMA,
        ],
    )
    def body(src_hbm, dst_hbm, x_hbm, out_hbm, sidx, didx, rows, agg, sem):
        cid = lax.axis_index("c")
        sid = lax.axis_index("s")
        wid = sid * NC + cid

        def zrow(r, _):
            def zcol(c, _):
                rows[r, pl.ds(c * 16, 16)] = jnp.zeros((16,), jnp.float32)
                return 0
            return lax.fori_loop(0, d // 16, zcol, 0)
        lax.fori_loop(0, CH, zrow, 0)

        def zcopy(t, _):
            pltpu.sync_copy(rows, agg.at[pl.ds(sid * rows_per_tile + t * CH, CH)])
            return 0
        lax.fori_loop(0, rows_per_tile // CH, zcopy, 0)
        plsc.subcore_barrier()

        base0 = wid * epw

        def step(j, _):
            base = base0 + j * CH
            pltpu.sync_copy(src_hbm.at[pl.ds(base, CH)], sidx)
            pltpu.sync_copy(dst_hbm.at[pl.ds(base, CH)], didx)
            pltpu.async_copy(x_hbm.at[sidx], rows, sem).wait()
            pltpu.sync_copy(rows, agg.at[didx], add=True)
            return 0
        lax.fori_loop(0, nchunk, step, 0)
        plsc.subcore_barrier()

        def wcopy(t, _):
            r0 = sid * rows_per_tile + t * CH
            pltpu.sync_copy(agg.at[pl.ds(r0, CH)], rows)
            pltpu.sync_copy(rows, out_hbm.at[cid, pl.ds(r0, CH)])
            return 0
        lax.fori_loop(0, rows_per_tile // CH, wcopy, 0)

    return body


def _mlp(x, a0, a1, W1, b1, W2, b2, eps, blk):
    n, d = x.shape

    def body(eps_ref, x_ref, a0_ref, a1_ref, w1_ref, b1_ref, w2_ref, b2_ref, o_ref):
        s = 1.0 + eps_ref[0, 0]
        out = s * x_ref[...] + a0_ref[...] + a1_ref[...]
        h = jnp.dot(out, w1_ref[...], preferred_element_type=jnp.float32)
        h = jnp.maximum(h + b1_ref[...], 0.0)
        o_ref[...] = jnp.dot(h, w2_ref[...], preferred_element_type=jnp.float32) + b2_ref[...]

    return pl.pallas_call(
        body,
        grid=(n // blk,),
        in_specs=[
            pl.BlockSpec(memory_space=pltpu.SMEM),
            pl.BlockSpec((blk, d), lambda i: (i, 0)),
            pl.BlockSpec((blk, d), lambda i: (i, 0)),
            pl.BlockSpec((blk, d), lambda i: (i, 0)),
            pl.BlockSpec((d, d), lambda i: (0, 0)),
            pl.BlockSpec((1, d), lambda i: (0, 0)),
            pl.BlockSpec((d, d), lambda i: (0, 0)),
            pl.BlockSpec((1, d), lambda i: (0, 0)),
        ],
        out_specs=pl.BlockSpec((blk, d), lambda i: (i, 0)),
        out_shape=jax.ShapeDtypeStruct((n, d), jnp.float32),
    )(eps, x, a0, a1, W1, b1.reshape(1, d), W2, b2.reshape(1, d))


def kernel(x, edge_index, W1, b1, W2, b2, eps):
    n, d = x.shape
    e = edge_index.shape[1]

    epw = -(-e // (NW * CH)) * CH
    e_pad = epw * NW
    n_pad = -(-(n + 1) // (NS * CH)) * (NS * CH)

    src = edge_index[0]
    dst = edge_index[1]
    pad = e_pad - e
    src_p = jnp.concatenate([src, jnp.zeros((pad,), jnp.int32)])
    dst_p = jnp.concatenate([dst, jnp.full((pad,), n, jnp.int32)])

    partial = _sc_aggregate(n_pad, d, epw)(src_p, dst_p, x)

    blk = 2000 if n % 2000 == 0 else (1250 if n % 1250 == 0 else n)
    return _mlp(x, partial[0, :n], partial[1, :n], W1, b1, W2, b2, eps, blk)
